# baseline (speedup 1.0000x reference)
import functools

import jax
import jax.numpy as jnp
from jax import lax
from jax.experimental import pallas as pl
from jax.experimental.pallas import tpu as pltpu
from jax.experimental.pallas import tpu_sc as plsc

_TT = 1024
_KC = 1024


def _argmin_body(xe_ref, em_ref, ei_ref):
    k = em_ref.shape[0]
    nkc = k // _KC
    xe = xe_ref[0]
    tt = xe.shape[1]

    def step(c, carry):
        mv, mi = carry
        em = em_ref[pl.ds(c * _KC, _KC), :]
        s = jnp.dot(em, xe, preferred_element_type=jnp.float32)
        lmv = jnp.min(s, axis=0, keepdims=True)
        iota = lax.broadcasted_iota(jnp.int32, s.shape, 0) + c * _KC
        lmi = jnp.min(jnp.where(s == lmv, iota, k), axis=0, keepdims=True)
        upd = lmv < mv
        return jnp.where(upd, lmv, mv), jnp.where(upd, lmi, mi)

    init = (jnp.full((1, tt), jnp.inf, jnp.float32),
            jnp.zeros((1, tt), jnp.int32))
    _, mi = lax.fori_loop(0, nkc, step, init)
    ei_ref[...] = mi.reshape(1, 1, tt)


def _vq_argmin(enc, e):
    b, c, hw = enc.shape
    k = e.shape[0]
    nt = hw // _TT
    esq = jnp.sum(e * e, axis=1, keepdims=True)
    em = jnp.concatenate([-2.0 * e, esq], axis=1)
    ones = jnp.ones((b, 1, hw), jnp.float32)
    xe = jnp.concatenate([enc, ones], axis=1)
    out = pl.pallas_call(
        _argmin_body,
        grid=(b, nt),
        in_specs=[
            pl.BlockSpec((1, c + 1, _TT), lambda i, j: (i, 0, j)),
            pl.BlockSpec((k, c + 1), lambda i, j: (0, 0)),
        ],
        out_specs=pl.BlockSpec((1, 1, _TT), lambda i, j: (i * nt + j, 0, 0)),
        out_shape=jax.ShapeDtypeStruct((b * nt, 1, _TT), jnp.int32),
    )(xe, em)
    return out.reshape(b * hw)


def _sc_gather(table, idx):
    info = plsc.get_sparse_core_info()
    nc, ns = info.num_cores, info.num_subcores
    nw = nc * ns
    b = idx.shape[0]
    d = table.shape[1]
    bpw = b // nw

    mesh = plsc.VectorSubcoreMesh(core_axis_name="c", subcore_axis_name="s")

    @functools.partial(
        pl.kernel,
        mesh=mesh,
        out_type=jax.ShapeDtypeStruct((b, d), jnp.float32),
        scratch_types=[
            pltpu.VMEM((bpw,), jnp.int32),
            pltpu.VMEM((bpw, d), jnp.float32),
            pltpu.SemaphoreType.DMA,
        ],
    )
    def gk(table_hbm, idx_hbm, out_hbm, idx_v, rows_v, sem):
        wid = lax.axis_index("s") * nc + lax.axis_index("c")
        base = wid * bpw
        pltpu.sync_copy(idx_hbm.at[pl.ds(base, bpw)], idx_v)
        pltpu.async_copy(table_hbm.at[idx_v], rows_v, sem).wait()
        pltpu.sync_copy(rows_v, out_hbm.at[pl.ds(base, bpw)])

    return gk(table, idx)


def _conv(x, w, b, stride, pad):
    x = jnp.pad(x, ((0, 0), (0, 0), (pad, pad), (pad, pad)))
    y = lax.conv_general_dilated(x, w, (stride, stride), 'VALID',
                                 dimension_numbers=('NCHW', 'OIHW', 'NCHW'))
    return y + b[None, :, None, None]


def _upsample2(x):
    return jnp.repeat(jnp.repeat(x, 2, axis=2), 2, axis=3)


def kernel(x, ew1, eb1, ew2, eb2, ew3, eb3, dw1, db1, dw2, db2, dw3, db3, E):
    h = jax.nn.relu(_conv(x, ew1, eb1, 2, 1))
    h = jax.nn.relu(_conv(h, ew2, eb2, 2, 1))
    enc = _conv(h, ew3, eb3, 2, 1)
    b, c, hh, ww = enc.shape
    ei = _vq_argmin(enc.reshape(b, c, hh * ww), E)
    ep = jnp.pad(E, ((0, 0), (0, 128 - c)))
    zq = _sc_gather(ep, ei)[:, :c]
    dec_in = zq.reshape(b, c, hh, ww)
    g = _upsample2(dec_in)
    g = jax.nn.relu(_conv(g, dw1, db1, 1, 1))
    g = _upsample2(g)
    g = jax.nn.relu(_conv(g, dw2, db2, 1, 1))
    g = _upsample2(g)
    g = _conv(g, dw3, db3, 1, 1)
    return jax.nn.sigmoid(g)


# speedup vs baseline: 2.1537x; 2.1537x over previous
import functools

import jax
import jax.numpy as jnp
from jax import lax
from jax.experimental import pallas as pl
from jax.experimental.pallas import tpu as pltpu
from jax.experimental.pallas import tpu_sc as plsc

_TT = 1024
_KC = 1024


def _argmin_body(xe_ref, em_ref, ei_ref):
    k = em_ref.shape[0]
    nkc = k // _KC
    xe = xe_ref[0]
    tt = xe.shape[1]

    def step(c, carry):
        mv, mi = carry
        em = em_ref[pl.ds(c * _KC, _KC), :]
        s = jnp.dot(em, xe, preferred_element_type=jnp.float32)
        lmv = jnp.min(s, axis=0, keepdims=True)
        iota = lax.broadcasted_iota(jnp.int32, s.shape, 0) + c * _KC
        lmi = jnp.min(jnp.where(s == lmv, iota, k), axis=0, keepdims=True)
        upd = lmv < mv
        return jnp.where(upd, lmv, mv), jnp.where(upd, lmi, mi)

    init = (jnp.full((1, tt), jnp.inf, jnp.float32),
            jnp.zeros((1, tt), jnp.int32))
    _, mi = lax.fori_loop(0, nkc, step, init)
    ei_ref[...] = mi.reshape(1, 1, tt)


def _vq_argmin(enc, e):
    b, c, hw = enc.shape
    k = e.shape[0]
    nt = hw // _TT
    esq = jnp.sum(e * e, axis=1, keepdims=True)
    em = jnp.concatenate([-2.0 * e, esq], axis=1)
    ones = jnp.ones((b, 1, hw), jnp.float32)
    xe = jnp.concatenate([enc, ones], axis=1)
    out = pl.pallas_call(
        _argmin_body,
        grid=(b, nt),
        in_specs=[
            pl.BlockSpec((1, c + 1, _TT), lambda i, j: (i, 0, j)),
            pl.BlockSpec((k, c + 1), lambda i, j: (0, 0)),
        ],
        out_specs=pl.BlockSpec((1, 1, _TT), lambda i, j: (i * nt + j, 0, 0)),
        out_shape=jax.ShapeDtypeStruct((b * nt, 1, _TT), jnp.int32),
    )(xe, em)
    return out.reshape(b * hw)


def _sc_gather(table, idx):
    info = plsc.get_sparse_core_info()
    nc, ns = info.num_cores, info.num_subcores
    nw = nc * ns
    b = idx.shape[0]
    d = table.shape[1]
    bpw = b // nw

    mesh = plsc.VectorSubcoreMesh(core_axis_name="c", subcore_axis_name="s")

    @functools.partial(
        pl.kernel,
        mesh=mesh,
        out_type=jax.ShapeDtypeStruct((b, d), jnp.float32),
        compiler_params=pltpu.CompilerParams(use_tc_tiling_on_sc=False),
        scratch_types=[
            pltpu.VMEM((bpw,), jnp.int32),
            pltpu.VMEM((bpw, d), jnp.float32),
            pltpu.SemaphoreType.DMA,
        ],
    )
    def gk(table_hbm, idx_hbm, out_hbm, idx_v, rows_v, sem):
        wid = lax.axis_index("s") * nc + lax.axis_index("c")
        base = wid * bpw
        pltpu.sync_copy(idx_hbm.at[pl.ds(base, bpw)], idx_v)
        pltpu.async_copy(table_hbm.at[idx_v], rows_v, sem).wait()
        pltpu.sync_copy(rows_v, out_hbm.at[pl.ds(base, bpw)])

    return gk(table, idx)


_ROWG = {0: ((0,), (1, 2)), 1: ((0, 1), (2,))}
_COLG = {0: ((0,), (1, 2)), 1: ((0, 1), (2,))}


def _phase_weights(w):
    ph = []
    for a in (0, 1):
        row = []
        for b in (0, 1):
            blocks = []
            for u in (0, 1):
                for v in (0, 1):
                    t = sum(w[:, :, kh, kw]
                            for kh in _ROWG[a][u] for kw in _COLG[b][v])
                    blocks.append(t)
            row.append(jnp.concatenate(blocks, axis=1))
        ph.append(jnp.stack(row))
    return jnp.stack(ph)


def _up_conv_body(act, xm_ref, x0_ref, xp_ref, w_ref, b_ref, out_ref):
    ci, w = x0_ref.shape[2], x0_ref.shape[3]
    i = pl.program_id(1)
    h = pl.num_programs(1)
    xm = xm_ref[0, 0] * jnp.where(i > 0, 1.0, 0.0)
    x0 = x0_ref[0, 0]
    xp = xp_ref[0, 0] * jnp.where(i < h - 1, 1.0, 0.0)
    zc = jnp.zeros((ci, 1), jnp.float32)

    def shl(x):
        return jnp.concatenate([zc, x[:, :-1]], axis=1)

    def shr(x):
        return jnp.concatenate([x[:, 1:], zc], axis=1)

    bias = b_ref[...]
    rows = ((xm, x0), (x0, xp))
    for a in (0, 1):
        u0, u1 = rows[a]
        ys = []
        for b in (0, 1):
            if b == 0:
                xs = jnp.concatenate([shl(u0), u0, shl(u1), u1], axis=0)
            else:
                xs = jnp.concatenate([u0, shr(u0), u1, shr(u1)], axis=0)
            ys.append(jnp.dot(w_ref[a, b], xs,
                              preferred_element_type=jnp.float32))
        co = ys[0].shape[0]
        q = lax.broadcasted_iota(jnp.int32, (co, 128), 1)
        perm = (q >> 1) + (q & 1) * 64
        chunks = []
        for kk in range(w // 64):
            cc = jnp.concatenate(
                [ys[0][:, 64 * kk:64 * (kk + 1)],
                 ys[1][:, 64 * kk:64 * (kk + 1)]], axis=1)
            chunks.append(jnp.take_along_axis(cc, perm, axis=1))
        row = jnp.concatenate(chunks, axis=1) + bias
        out_ref[0, a] = act(row)


def _up_conv(x, wc, bc, act):
    bsz, h, c, w = x.shape
    co = wc.shape[0]
    wst = _phase_weights(wc)
    body = functools.partial(_up_conv_body, act)
    return pl.pallas_call(
        body,
        grid=(bsz, h),
        in_specs=[
            pl.BlockSpec((1, 1, c, w), lambda n, i: (n, jnp.maximum(i - 1, 0), 0, 0)),
            pl.BlockSpec((1, 1, c, w), lambda n, i: (n, i, 0, 0)),
            pl.BlockSpec((1, 1, c, w),
                         lambda n, i: (n, jnp.minimum(i + 1, h - 1), 0, 0)),
            pl.BlockSpec((2, 2, co, 4 * c), lambda n, i: (0, 0, 0, 0)),
            pl.BlockSpec((co, 1), lambda n, i: (0, 0)),
        ],
        out_specs=pl.BlockSpec((1, 2, co, 2 * w), lambda n, i: (n, i, 0, 0)),
        out_shape=jax.ShapeDtypeStruct((bsz, 2 * h, co, 2 * w), jnp.float32),
    )(x, x, x, wst, bc.reshape(co, 1))


def _conv(x, w, b, stride, pad):
    x = jnp.pad(x, ((0, 0), (0, 0), (pad, pad), (pad, pad)))
    y = lax.conv_general_dilated(x, w, (stride, stride), 'VALID',
                                 dimension_numbers=('NCHW', 'OIHW', 'NCHW'))
    return y + b[None, :, None, None]


def kernel(x, ew1, eb1, ew2, eb2, ew3, eb3, dw1, db1, dw2, db2, dw3, db3, E):
    h = jax.nn.relu(_conv(x, ew1, eb1, 2, 1))
    h = jax.nn.relu(_conv(h, ew2, eb2, 2, 1))
    enc = _conv(h, ew3, eb3, 2, 1)
    b, c, hh, ww = enc.shape
    ei = _vq_argmin(enc.reshape(b, c, hh * ww), E)
    zq = _sc_gather(E, ei)
    dec_in = jnp.transpose(zq.reshape(b, c, hh, ww), (0, 2, 1, 3))
    g = _up_conv(dec_in, dw1, db1, jax.nn.relu)
    g = _up_conv(g, dw2, db2, jax.nn.relu)
    g = _up_conv(g, dw3, db3, jax.nn.sigmoid)
    return jnp.transpose(g, (0, 2, 1, 3))
